# baseline (speedup 1.0000x reference)
import jax
import jax.numpy as jnp
from jax.experimental import pallas as pl

HID = 64
HEADS = 4
C = HID // HEADS
L = 4
NQ = 25000


def _ln(x, g, b):
    mu = x.mean(-1, keepdims=True)
    v = ((x - mu) ** 2).mean(-1, keepdims=True)
    return (x - mu) / jnp.sqrt(v + 1e-5) * g + b


def _gat(x, src, dst, W, a_s, a_d, b):
    n = x.shape[0]
    h = (x @ W).reshape(n, HEADS, C)
    asrc = (h * a_s).sum(-1)
    adst = (h * a_d).sum(-1)
    alpha = asrc[src] + adst[dst]
    alpha = jax.nn.leaky_relu(alpha, 0.2)
    amax = jax.ops.segment_max(alpha, dst, num_segments=n)
    amax = jnp.where(jnp.isfinite(amax), amax, 0.0)
    ex = jnp.exp(alpha - amax[dst])
    den = jax.ops.segment_sum(ex, dst, num_segments=n)
    w = ex / (den[dst] + 1e-16)
    out = jax.ops.segment_sum(h[src] * w[:, :, None], dst, num_segments=n)
    return out.reshape(n, HID) + b


def _mlp_kernel(x_ref, w1_ref, b1_ref, w2_ref, b2_ref, w3_ref, b3_ref, o_ref):
    h = jnp.maximum(x_ref[...] @ w1_ref[...] + b1_ref[...], 0.0)
    h = jnp.maximum(h @ w2_ref[...] + b2_ref[...], 0.0)
    o_ref[...] = h @ w3_ref[...] + b3_ref[...]


def kernel(node_features, edge_index, emb_W, emb_b, emb_g, emb_be, gat_W, gat_as, gat_ad, gat_b, ln_g, ln_b, pW1, pb1, pW2, pb2, pW3, pb3, vW1, vb1, vW2, vb2, vW3, vb3):
    n = node_features.shape[0]
    x = jax.nn.relu(_ln(node_features @ emb_W + emb_b, emb_g, emb_be))
    loop = jnp.arange(n)
    src = jnp.concatenate([edge_index[0], loop])
    dst = jnp.concatenate([edge_index[1], loop])
    for i in range(L):
        x_new = jax.nn.relu(_gat(x, src, dst, gat_W[i], gat_as[i], gat_ad[i], gat_b[i]))
        x = _ln(x + x_new, ln_g[i], ln_b[i])
    dq = x[:NQ]
    logits = pl.pallas_call(
        _mlp_kernel,
        out_shape=jax.ShapeDtypeStruct((NQ, 2), jnp.float32),
    )(dq, pW1, pb1[None, :], pW2, pb2[None, :], pW3, pb3[None, :])
    pooled = x.mean(axis=0, keepdims=True)
    v = jax.nn.relu(pooled @ vW1 + vb1)
    v = jax.nn.relu(v @ vW2 + vb2)
    value = v @ vW3 + vb3
    return (logits, value, x)


# speedup vs baseline: 31.2415x; 31.2415x over previous
import jax
import jax.numpy as jnp
from jax import lax
from jax.experimental import pallas as pl
from jax.experimental.pallas import tpu as pltpu
from jax.experimental.pallas import tpu_sc as plsc

N = 50000
NP = 51200
E = 800000
EPAD = 819200
HID = 64
HEADS = 4
C = 16
L = 4
NQ = 25000
FD = 10

K = 512
SUB = 4
CHUNKS = 100
NH = NP // 2
RPT = NH // 16
NSTG = NP // 16

BLK = 5120
NBLK = NP // BLK


def _embed_body(nf_ref, w_ref, b_ref, g_ref, be_ref, x_ref):
    e = jnp.dot(nf_ref[...], w_ref[...], preferred_element_type=jnp.float32)
    e = e + b_ref[...]
    mu = e.mean(-1, keepdims=True)
    v = ((e - mu) ** 2).mean(-1, keepdims=True)
    y = (e - mu) / jnp.sqrt(v + 1e-5) * g_ref[...] + be_ref[...]
    x_ref[...] = jnp.maximum(y, 0.0)


def _embed(nf_pad, emb_W, emb_b, emb_g, emb_be):
    return pl.pallas_call(
        _embed_body,
        grid=(NBLK,),
        in_specs=[
            pl.BlockSpec((BLK, FD), lambda i: (i, 0)),
            pl.BlockSpec((FD, HID), lambda i: (0, 0)),
            pl.BlockSpec((1, HID), lambda i: (0, 0)),
            pl.BlockSpec((1, HID), lambda i: (0, 0)),
            pl.BlockSpec((1, HID), lambda i: (0, 0)),
        ],
        out_specs=pl.BlockSpec((BLK, HID), lambda i: (i, 0)),
        out_shape=jax.ShapeDtypeStruct((NP, HID), jnp.float32),
    )(nf_pad, emb_W, emb_b[None, :], emb_g[None, :], emb_be[None, :])


def _prep_body(x_ref, w_ref, wasT_ref, wadT_ref, h_ref, asrc_ref, adst_ref):
    x64 = x_ref[...]
    hh = jnp.dot(x64, w_ref[...], preferred_element_type=jnp.float32)
    h_ref[:, :HID] = hh
    h_ref[:, HID:] = jnp.zeros_like(hh)
    dn = (((1,), (1,)), ((), ()))
    asrc_ref[...] = lax.dot_general(wasT_ref[...], x64, dn,
                                    preferred_element_type=jnp.float32)
    adst_ref[...] = lax.dot_general(wadT_ref[...], x64, dn,
                                    preferred_element_type=jnp.float32)


def _prep(x, W, wasT, wadT):
    return pl.pallas_call(
        _prep_body,
        grid=(NBLK,),
        in_specs=[
            pl.BlockSpec((BLK, HID), lambda i: (i, 0)),
            pl.BlockSpec((HID, HID), lambda i: (0, 0)),
            pl.BlockSpec((HEADS, HID), lambda i: (0, 0)),
            pl.BlockSpec((HEADS, HID), lambda i: (0, 0)),
        ],
        out_specs=[
            pl.BlockSpec((BLK, 128), lambda i: (i, 0)),
            pl.BlockSpec((HEADS, BLK), lambda i: (0, i)),
            pl.BlockSpec((HEADS, BLK), lambda i: (0, i)),
        ],
        out_shape=[
            jax.ShapeDtypeStruct((NP, 128), jnp.float32),
            jax.ShapeDtypeStruct((HEADS, NP), jnp.float32),
            jax.ShapeDtypeStruct((HEADS, NP), jnp.float32),
        ],
    )(x, W, wasT, wadT)


def _epilogue_body(x_ref, h_ref, nd_ref, asrc_ref, adst_ref,
                   b_ref, g_ref, be_ref, xn_ref):
    sC = asrc_ref[...] + adst_ref[...]
    exsC = jnp.exp(jnp.maximum(sC, 0.2 * sC))
    exs64 = jnp.concatenate(
        [jnp.broadcast_to(exsC[k][:, None], (BLK, C)) for k in range(HEADS)],
        axis=1)
    nd = nd_ref[...]
    num64 = nd[:, :HID]
    den64 = nd[:, HID:]
    h64 = h_ref[:, :HID]
    numt = num64 + exs64 * h64
    dent = den64 + exs64
    out = numt / (dent + 1e-16)
    out = jnp.maximum(out + b_ref[...], 0.0)
    xr = x_ref[...] + out
    mu = xr.mean(-1, keepdims=True)
    d = xr - mu
    var = (d * d).mean(-1, keepdims=True)
    y = d / jnp.sqrt(var + 1e-5)
    xn_ref[...] = y * g_ref[...] + be_ref[...]


def _epilogue(x, h128, nd, asrc, adst, b, g, be):
    return pl.pallas_call(
        _epilogue_body,
        grid=(NBLK,),
        in_specs=[
            pl.BlockSpec((BLK, HID), lambda i: (i, 0)),
            pl.BlockSpec((BLK, 128), lambda i: (i, 0)),
            pl.BlockSpec((BLK, 128), lambda i: (i, 0)),
            pl.BlockSpec((HEADS, BLK), lambda i: (0, i)),
            pl.BlockSpec((HEADS, BLK), lambda i: (0, i)),
            pl.BlockSpec((1, HID), lambda i: (0, 0)),
            pl.BlockSpec((1, HID), lambda i: (0, 0)),
            pl.BlockSpec((1, HID), lambda i: (0, 0)),
        ],
        out_specs=pl.BlockSpec((BLK, HID), lambda i: (i, 0)),
        out_shape=jax.ShapeDtypeStruct((NP, HID), jnp.float32),
    )(x, h128, nd, asrc, adst, b[None, :], g[None, :], be[None, :])


def _pool_body(x_ref, psum_ref):
    i = pl.program_id(0)
    rid = lax.broadcasted_iota(jnp.int32, (BLK, 1), 0) + i * BLK
    xm = jnp.where(rid < N, x_ref[...], 0.0)
    part = xm.reshape(BLK // 8, 8, HID).sum(axis=0)

    @pl.when(i == 0)
    def _():
        psum_ref[...] = jnp.zeros_like(psum_ref)

    psum_ref[...] = psum_ref[...] + part


def _pool(x):
    return pl.pallas_call(
        _pool_body,
        grid=(NBLK,),
        in_specs=[pl.BlockSpec((BLK, HID), lambda i: (i, 0))],
        out_specs=pl.BlockSpec((8, HID), lambda i: (0, 0)),
        out_shape=jax.ShapeDtypeStruct((8, HID), jnp.float32),
    )(x)


def _logits_body(x_ref, w1, b1, w2, b2, w3, b3, o_ref):
    h = jnp.maximum(jnp.dot(x_ref[...], w1[...],
                            preferred_element_type=jnp.float32) + b1[...], 0.0)
    h = jnp.maximum(jnp.dot(h, w2[...],
                            preferred_element_type=jnp.float32) + b2[...], 0.0)
    o_ref[...] = jnp.dot(h, w3[...], preferred_element_type=jnp.float32) + b3[...]


def _logits(dq, pW1, pb1, pW2, pb2, pW3, pb3):
    QB = 5000
    return pl.pallas_call(
        _logits_body,
        grid=(NQ // QB,),
        in_specs=[
            pl.BlockSpec((QB, HID), lambda i: (i, 0)),
            pl.BlockSpec((HID, HID), lambda i: (0, 0)),
            pl.BlockSpec((1, HID), lambda i: (0, 0)),
            pl.BlockSpec((HID, 32), lambda i: (0, 0)),
            pl.BlockSpec((1, 32), lambda i: (0, 0)),
            pl.BlockSpec((32, 2), lambda i: (0, 0)),
            pl.BlockSpec((1, 2), lambda i: (0, 0)),
        ],
        out_specs=pl.BlockSpec((QB, 2), lambda i: (i, 0)),
        out_shape=jax.ShapeDtypeStruct((NQ, 2), jnp.float32),
    )(dq, pW1, pb1[None, :], pW2, pb2[None, :], pW3, pb3[None, :])


def _value_body(ps_ref, w1, b1, w2, b2, w3, b3, o_ref):
    pooled = jnp.sum(ps_ref[...], axis=0, keepdims=True) * (1.0 / N)
    v = jnp.maximum(jnp.dot(pooled, w1[...],
                            preferred_element_type=jnp.float32) + b1[...], 0.0)
    v = jnp.maximum(jnp.dot(v, w2[...],
                            preferred_element_type=jnp.float32) + b2[...], 0.0)
    o_ref[...] = jnp.dot(v, w3[...], preferred_element_type=jnp.float32) + b3[...]


def _value(psum, vW1, vb1, vW2, vb2, vW3, vb3):
    return pl.pallas_call(
        _value_body,
        out_shape=jax.ShapeDtypeStruct((1, 1), jnp.float32),
    )(psum, vW1, vb1[None, :], vW2, vb2[None, :], vW3, vb3[None, :])


def _edge_body(src_hbm, dst_hbm, h128_hbm, asrc_hbm, adst_hbm,
               nd_out, hlin_out,
               asrc_tab, adst_tab, src_v, dst_v, sadj,
               hrows, ex_v, stg, stg16, zblk, zrow, drep, dslc, sem,
               num_acc, den_acc):
    c = lax.axis_index("c")
    s = lax.axis_index("s")

    sbase = s * NSTG

    def _stage(k, _):
        r0 = sbase + k * 32
        pltpu.sync_copy(h128_hbm.at[pl.ds(r0, 32)], stg)
        for r in range(32):
            for hh in range(HEADS):
                stg16[r * HEADS + hh, :] = stg[r, pl.ds(hh * C, C)]
        pltpu.sync_copy(stg16, hlin_out.at[pl.ds(r0 * HEADS, 32 * HEADS)])
        return 0
    lax.fori_loop(0, NSTG // 32, _stage, 0)

    for i in range(4):
        zrow[pl.ds(i * 16, 16)] = jnp.zeros((16,), jnp.float32)

    def _zb(r, _):
        zblk[r, :] = jnp.zeros((C,), jnp.float32)
        return 0
    lax.fori_loop(0, 64, _zb, 0)
    plsc.subcore_barrier()

    def _pass(pp, _carry):
        head = (pp // 2) * 2 + c
        headN = head * NP
        lo = (pp % 2) * NH

        pltpu.sync_copy(asrc_hbm.at[pl.ds(headN, NP)], asrc_tab)
        pltpu.sync_copy(adst_hbm.at[pl.ds(headN + lo, NH)], adst_tab)

        def _zcp(k, _):
            o = s * RPT + k * 64
            pltpu.sync_copy(zblk, num_acc.at[pl.ds(o, 64)])
            pltpu.sync_copy(zrow, den_acc.at[pl.ds(o, 64)])
            return 0
        lax.fori_loop(0, RPT // 64, _zcp, 0)
        plsc.subcore_barrier()

        def _chunk(j, _):
            cg = s * CHUNKS + j
            pltpu.sync_copy(src_hbm.at[pl.ds(cg * SUB, SUB)], src_v)
            pltpu.sync_copy(dst_hbm.at[pl.ds(cg * SUB, SUB)], dst_v)
            ebase = cg * K
            for jj in range(SUB):
                for i in range(8):
                    sl = pl.ds(i * 16, 16)
                    sv = src_v[jj, sl]
                    dv = dst_v[jj, sl]
                    dl = dv - lo
                    okd = (dl >= 0) & (dl < NH)
                    dli = jnp.where(okd, dl, 0)
                    av = plsc.load_gather(asrc_tab, [sv])
                    bv = plsc.load_gather(adst_tab, [dli])
                    al = av + bv
                    al = jnp.maximum(al, 0.2 * al)
                    ev = jnp.exp(al)
                    gidx = lax.iota(jnp.int32, 16) + (ebase + jj * 128 + i * 16)
                    ok = (gidx < E) & okd
                    ev = jnp.where(ok, ev, 0.0)
                    ex_v[jj, sl] = ev
                    sadj[jj, sl] = sv * HEADS + head
                    dst_v[jj, sl] = dli
            for jj in range(SUB):
                pltpu.async_copy(hlin_out.at[sadj.at[jj]],
                                 hrows.at[pl.ds(jj * 128, 128)], sem).wait()
            for g in range(K // 16):
                ev16 = ex_v[g // 8, pl.ds((g % 8) * 16, 16)]
                for l in range(16):
                    e = g * 16 + l
                    hrows[e, :] = hrows[e, :] * ev16[l]
            for jj in range(SUB):
                pltpu.sync_copy(hrows.at[pl.ds(jj * 128, 128)],
                                num_acc.at[dst_v.at[jj]], add=True)
                pltpu.sync_copy(ex_v.at[jj], den_acc.at[dst_v.at[jj]], add=True)
            return 0
        lax.fori_loop(0, CHUNKS, _chunk, 0)
        plsc.subcore_barrier()

        def _wb(k, _):
            o = s * RPT + k * 64
            pltpu.sync_copy(num_acc.at[pl.ds(o, 64)],
                            nd_out.at[pl.ds(lo + o, 64), pl.ds(head * C, C)])
            pltpu.sync_copy(den_acc.at[pl.ds(o, 64)], dslc)
            for i in range(4):
                dv16 = dslc[pl.ds(i * 16, 16)]
                for l in range(16):
                    drep[i * 16 + l, :] = jnp.full((C,), dv16[l],
                                                   dtype=jnp.float32)
            pltpu.sync_copy(drep,
                            nd_out.at[pl.ds(lo + o, 64),
                                      pl.ds(HID + head * C, C)])
            return 0
        lax.fori_loop(0, RPT // 64, _wb, 0)
        plsc.subcore_barrier()
        return 0

    lax.fori_loop(0, 4, _pass, 0)


def _make_edge_kernel():
    return pl.kernel(
        _edge_body,
        out_type=[
            jax.ShapeDtypeStruct((NP, 128), jnp.float32),
            jax.ShapeDtypeStruct((HEADS * NP, C), jnp.float32),
        ],
        mesh=plsc.VectorSubcoreMesh(core_axis_name="c", subcore_axis_name="s"),
        compiler_params=pltpu.CompilerParams(needs_layout_passes=False,
                                             use_tc_tiling_on_sc=False),
        scratch_types=[
            pltpu.VMEM((NP,), jnp.float32),
            pltpu.VMEM((NH,), jnp.float32),
            pltpu.VMEM((SUB, 128), jnp.int32),
            pltpu.VMEM((SUB, 128), jnp.int32),
            pltpu.VMEM((SUB, 128), jnp.int32),
            pltpu.VMEM((K, C), jnp.float32),
            pltpu.VMEM((SUB, 128), jnp.float32),
            pltpu.VMEM((32, 128), jnp.float32),
            pltpu.VMEM((32 * HEADS, C), jnp.float32),
            pltpu.VMEM((64, C), jnp.float32),
            pltpu.VMEM((64,), jnp.float32),
            pltpu.VMEM((64, C), jnp.float32),
            pltpu.VMEM((64,), jnp.float32),
            pltpu.SemaphoreType.DMA,
            pltpu.VMEM_SHARED((NH, C), jnp.float32),
            pltpu.VMEM_SHARED((NH,), jnp.float32),
        ],
    )


_edge_kernel = _make_edge_kernel()


def kernel(node_features, edge_index, emb_W, emb_b, emb_g, emb_be, gat_W,
           gat_as, gat_ad, gat_b, ln_g, ln_b, pW1, pb1, pW2, pb2, pW3, pb3,
           vW1, vb1, vW2, vb2, vW3, vb3):
    pad = (jnp.arange(EPAD - E, dtype=jnp.int32) * 97) % N
    src = jnp.concatenate([edge_index[0].astype(jnp.int32), pad])
    dst = jnp.concatenate([edge_index[1].astype(jnp.int32), pad])
    src = src.reshape(EPAD // 128, 128)
    dst = dst.reshape(EPAD // 128, 128)
    nf_pad = jnp.pad(node_features, ((0, NP - N), (0, 0)))

    x = _embed(nf_pad, emb_W, emb_b, emb_g, emb_be)

    for i in range(L):
        W = gat_W[i]
        wasT = jnp.einsum('fkc,kc->kf', W.reshape(HID, HEADS, C), gat_as[i])
        wadT = jnp.einsum('fkc,kc->kf', W.reshape(HID, HEADS, C), gat_ad[i])
        h128, asrc, adst = _prep(x, W, wasT, wadT)
        nd, _hlin = _edge_kernel(src, dst, h128,
                                 asrc.reshape(HEADS * NP),
                                 adst.reshape(HEADS * NP))
        x = _epilogue(x, h128, nd, asrc, adst, gat_b[i], ln_g[i], ln_b[i])

    psum = _pool(x)
    xout = x[:N]
    logits = _logits(x[:NQ], pW1, pb1, pW2, pb2, pW3, pb3)
    value = _value(psum, vW1, vb1, vW2, vb2, vW3, vb3)
    return (logits, value, xout)
